# baseline (speedup 1.0000x reference)
import functools

import jax
import jax.numpy as jnp
from jax import lax
from jax.experimental import pallas as pl
from jax.experimental.pallas import tpu as pltpu
from jax.experimental.pallas import tpu_sc as plsc

_S = 16
_CH = 16
_L = 16


def _sc_pool(x_hbm, out_hbm, buf, acc, sem):
    nseg_total, seg_rows, h = x_hbm.shape
    nch = seg_rows // _CH
    nj = h // _L
    nc = 2
    wid = lax.axis_index("s") * nc + lax.axis_index("c")

    def start(seg, c, slot):
        pltpu.make_async_copy(
            x_hbm.at[seg, pl.ds(c * _CH, _CH)], buf.at[slot], sem.at[slot]
        ).start()

    for segi in range(2):
        seg = wid * 2 + segi
        start(seg, 0, 0)
        for c in range(nch):
            slot = c % 2
            if c + 1 < nch:
                start(seg, c + 1, (c + 1) % 2)
            pltpu.make_async_copy(
                x_hbm.at[seg, pl.ds(c * _CH, _CH)], buf.at[slot], sem.at[slot]
            ).wait()
            first = c == 0

            def jbody(j, _, slot=slot, first=first):
                off = pl.multiple_of(j * _L, _L)
                a = buf[slot, 0, pl.ds(off, _L)]
                for r in range(1, _CH):
                    a = a + buf[slot, r, pl.ds(off, _L)]
                if first:
                    acc[pl.ds(off, _L)] = a
                else:
                    acc[pl.ds(off, _L)] = acc[pl.ds(off, _L)] + a
                return 0

            lax.fori_loop(0, nj, jbody, 0, unroll=False)

        def sbody(j, _):
            off = pl.multiple_of(j * _L, _L)
            acc[pl.ds(off, _L)] = acc[pl.ds(off, _L)] * (1.0 / seg_rows)
            return 0

        lax.fori_loop(0, nj, sbody, 0, unroll=False)
        pltpu.make_async_copy(acc, out_hbm.at[seg], sem.at[0]).start()
        pltpu.make_async_copy(acc, out_hbm.at[seg], sem.at[0]).wait()


def kernel(hidden_states, attention_mask):
    B, T, H = hidden_states.shape
    seg_rows = T // _S
    x = hidden_states.reshape(B * _S, seg_rows, H)
    mesh = plsc.VectorSubcoreMesh(core_axis_name="c", subcore_axis_name="s")
    f = pl.kernel(
        _sc_pool,
        mesh=mesh,
        out_type=jax.ShapeDtypeStruct((B * _S, H), hidden_states.dtype),
        scratch_types=[
            pltpu.VMEM((2, _CH, H), hidden_states.dtype),
            pltpu.VMEM((H,), hidden_states.dtype),
            pltpu.SemaphoreType.DMA((2,)),
        ],
    )
    seg_states = f(x).reshape(B, _S, H)
    seg_mask = jnp.ones((B, _S), dtype=jnp.bool_)
    return seg_states, seg_mask


# speedup vs baseline: 1.1065x; 1.1065x over previous
import functools

import jax
import jax.numpy as jnp
from jax import lax
from jax.experimental import pallas as pl
from jax.experimental.pallas import tpu as pltpu
from jax.experimental.pallas import tpu_sc as plsc

_S = 16
_CH = 16
_L = 16


def _sc_pool(x_hbm, out_hbm, buf, acc, sem):
    nseg_total, seg_rows, h = x_hbm.shape
    nch = seg_rows // _CH
    nj = h // _L
    nc = 2
    wid = lax.axis_index("s") * nc + lax.axis_index("c")

    def start(seg, c, slot):
        pltpu.make_async_copy(
            x_hbm.at[seg, pl.ds(c * _CH, _CH)], buf.at[slot], sem.at[slot]
        ).start()

    for segi in range(2):
        seg = wid * 2 + segi
        start(seg, 0, 0)
        for c in range(nch):
            slot = c % 2
            if c + 1 < nch:
                start(seg, c + 1, (c + 1) % 2)
            pltpu.make_async_copy(
                x_hbm.at[seg, pl.ds(c * _CH, _CH)], buf.at[slot], sem.at[slot]
            ).wait()
            first = c == 0

            def jbody(j, _, slot=slot, first=first):
                off = pl.multiple_of(j * _L, _L)
                vals = [buf[slot, r, pl.ds(off, _L)] for r in range(_CH)]
                while len(vals) > 1:
                    vals = [vals[i] + vals[i + 1] for i in range(0, len(vals), 2)]
                if first:
                    acc[pl.ds(off, _L)] = vals[0]
                else:
                    acc[pl.ds(off, _L)] = acc[pl.ds(off, _L)] + vals[0]
                return 0

            lax.fori_loop(0, nj, jbody, 0, unroll=2)

        def sbody(j, _):
            off = pl.multiple_of(j * _L, _L)
            acc[pl.ds(off, _L)] = acc[pl.ds(off, _L)] * (1.0 / seg_rows)
            return 0

        lax.fori_loop(0, nj, sbody, 0, unroll=False)
        pltpu.make_async_copy(acc, out_hbm.at[seg], sem.at[0]).start()
        pltpu.make_async_copy(acc, out_hbm.at[seg], sem.at[0]).wait()


def kernel(hidden_states, attention_mask):
    B, T, H = hidden_states.shape
    seg_rows = T // _S
    x = hidden_states.reshape(B * _S, seg_rows, H)
    mesh = plsc.VectorSubcoreMesh(core_axis_name="c", subcore_axis_name="s")
    f = pl.kernel(
        _sc_pool,
        mesh=mesh,
        out_type=jax.ShapeDtypeStruct((B * _S, H), hidden_states.dtype),
        scratch_types=[
            pltpu.VMEM((2, _CH, H), hidden_states.dtype),
            pltpu.VMEM((H,), hidden_states.dtype),
            pltpu.SemaphoreType.DMA((2,)),
        ],
    )
    seg_states = f(x).reshape(B, _S, H)
    seg_mask = jnp.ones((B, _S), dtype=jnp.bool_)
    return seg_states, seg_mask
